# baseline (speedup 1.0000x reference)
import functools

import jax
import jax.numpy as jnp
from jax import lax
from jax.experimental import pallas as pl
from jax.experimental.pallas import tpu as pltpu
from jax.experimental.pallas import tpu_sc as plsc

B, N = 4, 4096
D = 768
H = 768
M = B * N
POS = 10240
NC, NS = 2, 16
NW = NC * NS
C = 64
BM = 1024
GROUPS = (8192, 8192)


@functools.cache
def _pe_gather_kernel(mg):
    mpw = mg // NW
    nchunk = mpw // C
    ipw = mpw * 2
    mesh = plsc.VectorSubcoreMesh(core_axis_name="c", subcore_axis_name="s")

    @functools.partial(
        pl.kernel,
        mesh=mesh,
        out_type=jax.ShapeDtypeStruct((mg, H), jnp.float32),
        scratch_types=[
            pltpu.VMEM((ipw,), jnp.int32),
            pltpu.VMEM((2 * C, H), jnp.float32),
            pltpu.SemaphoreType.DMA,
        ],
    )
    def k(tab_hbm, idx_hbm, out_hbm, idxv, rows, sem):
        wid = lax.axis_index("s") * NC + lax.axis_index("c")
        pltpu.sync_copy(idx_hbm.at[pl.ds(wid * ipw, ipw)], idxv)

        def chunk(j, carry):
            pltpu.async_copy(
                tab_hbm.at[idxv.at[pl.ds(j * 2 * C, 2 * C)]], rows, sem
            ).wait()

            def add_row(r, c2):
                for c in range(H // 16):
                    sl = pl.ds(c * 16, 16)
                    rows[r, sl] = rows[r, sl] + rows[C + r, sl]
                return c2

            lax.fori_loop(0, C, add_row, 0)
            off = wid * mpw + j * C
            pltpu.sync_copy(rows.at[pl.ds(0, C)], out_hbm.at[pl.ds(off, C)])
            return carry

        lax.fori_loop(0, nchunk, chunk, 0)

    return k


def _proj(px_ref, w_ref):
    pxn = 2.0 * px_ref[...] - 1.0
    return lax.dot_general(
        pxn,
        w_ref[...],
        (((1,), (1,)), ((), ())),
        preferred_element_type=jnp.float32,
        precision=lax.Precision.DEFAULT,
    )


def _mm_body(px_ref, w_ref, pe_ref, out_ref):
    out_ref[...] = _proj(px_ref, w_ref) + pe_ref[...]


def _mm_body_acc(px_ref, w_ref, pe_ref, h_ref, out_ref):
    del h_ref
    _mm_body(px_ref, w_ref, pe_ref, out_ref)


def _mm_body_nope(px_ref, w_ref, out_ref):
    out_ref[...] = _proj(px_ref, w_ref)


def _add_body(pe_ref, h_ref, out_ref):
    out_ref[...] = h_ref[...] + pe_ref[...]


def _mm_prefill(tok0, mg, px, w):
    b0 = tok0 // BM
    return pl.pallas_call(
        _mm_body_nope,
        grid=(mg // BM,),
        in_specs=[
            pl.BlockSpec((BM, D), lambda i, b0=b0: (b0 + i, 0)),
            pl.BlockSpec((H, D), lambda i: (0, 0)),
        ],
        out_specs=pl.BlockSpec((BM, H), lambda i, b0=b0: (b0 + i, 0)),
        out_shape=jax.ShapeDtypeStruct((M, H), jnp.float32),
    )(px, w)


def _add_group(tok0, mg, pe_g, h):
    b0 = tok0 // BM
    return pl.pallas_call(
        _add_body,
        grid=(mg // BM,),
        in_specs=[
            pl.BlockSpec((BM, H), lambda i: (i, 0)),
            pl.BlockSpec((BM, H), lambda i, b0=b0: (b0 + i, 0)),
        ],
        out_specs=pl.BlockSpec((BM, H), lambda i, b0=b0: (b0 + i, 0)),
        out_shape=jax.ShapeDtypeStruct((M, H), jnp.float32),
        input_output_aliases={1: 0},
    )(pe_g, h)


def _mm_group(tok0, mg, px, w, pe_g, h):
    b0 = tok0 // BM
    out_spec = pl.BlockSpec((BM, H), lambda i, b0=b0: (b0 + i, 0))
    in_specs = [
        pl.BlockSpec((BM, D), lambda i, b0=b0: (b0 + i, 0)),
        pl.BlockSpec((H, D), lambda i: (0, 0)),
        pl.BlockSpec((BM, H), lambda i: (i, 0)),
    ]
    if h is None:
        return pl.pallas_call(
            _mm_body,
            grid=(mg // BM,),
            in_specs=in_specs,
            out_specs=out_spec,
            out_shape=jax.ShapeDtypeStruct((M, H), jnp.float32),
        )(px, w, pe_g)
    return pl.pallas_call(
        _mm_body_acc,
        grid=(mg // BM,),
        in_specs=in_specs + [pl.BlockSpec(memory_space=pl.ANY)],
        out_specs=out_spec,
        out_shape=jax.ShapeDtypeStruct((M, H), jnp.float32),
        input_output_aliases={3: 0},
    )(px, w, pe_g, h)


def kernel(pixel_values, pixel_position_ids, padding_mask, W, pos_table):
    del padding_mask
    px = pixel_values.reshape(M, D)
    table2 = pos_table.reshape(2 * POS, H)
    ids = pixel_position_ids.reshape(M, 2)
    ix = ids[:, 0].reshape(M // C, C)
    iy = ids[:, 1].reshape(M // C, C) + POS
    idx2 = jnp.stack([ix, iy], axis=1).reshape(2 * M)
    starts = [sum(GROUPS[:g]) for g in range(len(GROUPS))]
    pes = []
    for tok0, mg in zip(starts, GROUPS):
        idx_g = lax.slice(idx2, (tok0 * 2,), ((tok0 + mg) * 2,))
        pes.append(_pe_gather_kernel(mg)(table2, idx_g))
    h = _mm_prefill(starts[-1], GROUPS[-1], px, W)
    for tok0, mg, pe_g in zip(starts[:-1], GROUPS[:-1], pes[:-1]):
        h = _mm_group(tok0, mg, px, W, pe_g, h)
    h = _add_group(starts[-1], GROUPS[-1], pes[-1], h)
    return h.reshape(B, N, H)


# speedup vs baseline: 1.0536x; 1.0536x over previous
import functools

import jax
import jax.numpy as jnp
from jax import lax
from jax.experimental import pallas as pl
from jax.experimental.pallas import tpu as pltpu
from jax.experimental.pallas import tpu_sc as plsc

B, N = 4, 4096
D = 768
H = 768
M = B * N
POS = 10240
NC, NS = 2, 16
NW = NC * NS
C = 64
BM = 1024
GROUPS = (4096, 8192, 4096)


@functools.cache
def _pe_gather_kernel(mg):
    mpw = mg // NW
    nchunk = mpw // C
    ipw = mpw * 2
    mesh = plsc.VectorSubcoreMesh(core_axis_name="c", subcore_axis_name="s")

    @functools.partial(
        pl.kernel,
        mesh=mesh,
        out_type=jax.ShapeDtypeStruct((mg, H), jnp.float32),
        scratch_types=[
            pltpu.VMEM((ipw,), jnp.int32),
            pltpu.VMEM((2 * C, H), jnp.float32),
            pltpu.SemaphoreType.DMA,
        ],
    )
    def k(tab_hbm, idx_hbm, out_hbm, idxv, rows, sem):
        wid = lax.axis_index("s") * NC + lax.axis_index("c")
        pltpu.sync_copy(idx_hbm.at[pl.ds(wid * ipw, ipw)], idxv)

        def chunk(j, carry):
            pltpu.async_copy(
                tab_hbm.at[idxv.at[pl.ds(j * 2 * C, 2 * C)]], rows, sem
            ).wait()

            def add_row(r, c2):
                for c in range(H // 16):
                    sl = pl.ds(c * 16, 16)
                    rows[r, sl] = rows[r, sl] + rows[C + r, sl]
                return c2

            lax.fori_loop(0, C, add_row, 0)
            off = wid * mpw + j * C
            pltpu.sync_copy(rows.at[pl.ds(0, C)], out_hbm.at[pl.ds(off, C)])
            return carry

        lax.fori_loop(0, nchunk, chunk, 0)

    return k


def _proj(px_ref, w_ref):
    pxn = 2.0 * px_ref[...] - 1.0
    return lax.dot_general(
        pxn,
        w_ref[...],
        (((1,), (1,)), ((), ())),
        preferred_element_type=jnp.float32,
        precision=lax.Precision.DEFAULT,
    )


def _mm_body(px_ref, w_ref, pe_ref, out_ref):
    out_ref[...] = _proj(px_ref, w_ref) + pe_ref[...]


def _mm_body_acc(px_ref, w_ref, pe_ref, h_ref, out_ref):
    del h_ref
    _mm_body(px_ref, w_ref, pe_ref, out_ref)


def _mm_body_nope(px_ref, w_ref, out_ref):
    out_ref[...] = _proj(px_ref, w_ref)


def _add_body(pe_ref, h_ref, out_ref):
    out_ref[...] = h_ref[...] + pe_ref[...]


def _mm_prefill(tok0, mg, px, w):
    b0 = tok0 // BM
    return pl.pallas_call(
        _mm_body_nope,
        grid=(mg // BM,),
        in_specs=[
            pl.BlockSpec((BM, D), lambda i, b0=b0: (b0 + i, 0)),
            pl.BlockSpec((H, D), lambda i: (0, 0)),
        ],
        out_specs=pl.BlockSpec((BM, H), lambda i, b0=b0: (b0 + i, 0)),
        out_shape=jax.ShapeDtypeStruct((M, H), jnp.float32),
    )(px, w)


def _add_group(tok0, mg, pe_g, h):
    b0 = tok0 // BM
    return pl.pallas_call(
        _add_body,
        grid=(mg // BM,),
        in_specs=[
            pl.BlockSpec((BM, H), lambda i: (i, 0)),
            pl.BlockSpec((BM, H), lambda i, b0=b0: (b0 + i, 0)),
        ],
        out_specs=pl.BlockSpec((BM, H), lambda i, b0=b0: (b0 + i, 0)),
        out_shape=jax.ShapeDtypeStruct((M, H), jnp.float32),
        input_output_aliases={1: 0},
    )(pe_g, h)


def _mm_group(tok0, mg, px, w, pe_g, h):
    b0 = tok0 // BM
    out_spec = pl.BlockSpec((BM, H), lambda i, b0=b0: (b0 + i, 0))
    in_specs = [
        pl.BlockSpec((BM, D), lambda i, b0=b0: (b0 + i, 0)),
        pl.BlockSpec((H, D), lambda i: (0, 0)),
        pl.BlockSpec((BM, H), lambda i: (i, 0)),
    ]
    if h is None:
        return pl.pallas_call(
            _mm_body,
            grid=(mg // BM,),
            in_specs=in_specs,
            out_specs=out_spec,
            out_shape=jax.ShapeDtypeStruct((M, H), jnp.float32),
        )(px, w, pe_g)
    return pl.pallas_call(
        _mm_body_acc,
        grid=(mg // BM,),
        in_specs=in_specs + [pl.BlockSpec(memory_space=pl.ANY)],
        out_specs=out_spec,
        out_shape=jax.ShapeDtypeStruct((M, H), jnp.float32),
        input_output_aliases={3: 0},
    )(px, w, pe_g, h)


def kernel(pixel_values, pixel_position_ids, padding_mask, W, pos_table):
    del padding_mask
    px = pixel_values.reshape(M, D)
    table2 = pos_table.reshape(2 * POS, H)
    ids = pixel_position_ids.reshape(M, 2)
    ix = ids[:, 0].reshape(M // C, C)
    iy = ids[:, 1].reshape(M // C, C) + POS
    idx2 = jnp.stack([ix, iy], axis=1).reshape(2 * M)
    starts = [sum(GROUPS[:g]) for g in range(len(GROUPS))]
    pes = []
    for tok0, mg in zip(starts, GROUPS):
        idx_g = lax.slice(idx2, (tok0 * 2,), ((tok0 + mg) * 2,))
        pes.append(_pe_gather_kernel(mg)(table2, idx_g))
    h = None
    for tok0, mg, pe_g in zip(starts, GROUPS, pes):
        h = _mm_group(tok0, mg, px, W, pe_g, h)
    return h.reshape(B, N, H)
